# baseline (speedup 1.0000x reference)
import jax
import jax.numpy as jnp
import numpy as np
from jax.experimental import pallas as pl

B, N, D, H = 64, 1024, 192, 48
BB = 8
G = B // BB


def _np_uniform_f32(seed: int, size: int) -> np.ndarray:
    def rotl(x, r):
        return ((x << np.uint32(r)) | (x >> np.uint32(32 - r))).astype(np.uint32)

    k1 = np.uint32(np.uint64(seed) >> np.uint64(32))
    k2 = np.uint32(seed & 0xFFFFFFFF)
    rots = [(13, 15, 26, 6), (17, 29, 16, 24)]
    ks = [k1, k2, np.uint32(k1 ^ k2 ^ np.uint32(0x1BD11BDA))]
    with np.errstate(over="ignore"):
        x0 = np.zeros(size, np.uint32) + ks[0]
        x1 = (np.arange(size, dtype=np.uint32) + ks[1]).astype(np.uint32)
        for i in range(5):
            for r in rots[i % 2]:
                x0 = (x0 + x1).astype(np.uint32)
                x1 = (x0 ^ rotl(x1, r)).astype(np.uint32)
            x0 = (x0 + ks[(i + 1) % 3]).astype(np.uint32)
            x1 = (x1 + ks[(i + 2) % 3] + np.uint32(i + 1)).astype(np.uint32)
    bits = x0 ^ x1
    fb = (bits >> np.uint32(9)) | np.uint32(0x3F800000)
    return np.maximum(np.float32(0.0), fb.view(np.float32) - np.float32(1.0))


_U = _np_uniform_f32(42, B * N)
with np.errstate(divide="ignore"):
    _LU = (np.log(_U) - np.log1p(-_U)).astype(np.float32).reshape(G, BB, N)


def _body(x_ref, lu_ref, pk_ref, w1_ref, out_ref, m_ref):
    g = pl.program_id(0)
    xb = x_ref[...]
    w1 = w1_ref[...]
    mt = pk_ref[0:D]
    b1 = pk_ref[D:D + H]
    b2 = pk_ref[D + H:D + H + 1]
    w2c = pk_ref[D + H + 1:D + 2 * H + 1]
    rows = []
    for b in range(BB):
        ht = jnp.dot(w1, xb[b], preferred_element_type=jnp.float32)
        gl = jax.nn.gelu(ht + b1)
        rows.append(jax.lax.dot_general(
            w2c, gl, (((0,), (0,)), ((), ())),
            preferred_element_type=jnp.float32))
    s = jnp.concatenate(rows, axis=0) + b2 - lu_ref[g]
    cond = s > 0
    m_ref[g] = cond
    out_ref[...] = jnp.where(cond[:, None, :], mt[None], xb)


def kernel(x, mask_token, W1, b1, W2, b2):
    xt = jnp.transpose(x, (0, 2, 1))
    pk = jnp.concatenate([mask_token.reshape(D), b1, b2,
                          W2.reshape(H)]).reshape(D + 2 * H + 1, 1)
    outt, m = pl.pallas_call(
        _body,
        grid=(G,),
        in_specs=[
            pl.BlockSpec((BB, D, N), lambda g: (g, 0, 0)),
            pl.BlockSpec((G, BB, N), lambda g: (0, 0, 0)),
            pl.BlockSpec((D + 2 * H + 1, 1), lambda g: (0, 0)),
            pl.BlockSpec((H, D), lambda g: (0, 0)),
        ],
        out_specs=[
            pl.BlockSpec((BB, D, N), lambda g: (g, 0, 0)),
            pl.BlockSpec((G, BB, N), lambda g: (0, 0, 0)),
        ],
        out_shape=[
            jax.ShapeDtypeStruct((B, D, N), jnp.float32),
            jax.ShapeDtypeStruct((G, BB, N), jnp.bool_),
        ],
    )(xt, jnp.asarray(_LU), pk, W1)
    masked = jnp.transpose(outt, (0, 2, 1))
    return masked, m.reshape(B, N)


# speedup vs baseline: 1.0040x; 1.0040x over previous
import jax
import jax.numpy as jnp
import numpy as np
from jax.experimental import pallas as pl

B, N, D, H = 64, 1024, 192, 48
BB = 8
G = B // BB


def _np_uniform_f32(seed: int, size: int) -> np.ndarray:
    def rotl(x, r):
        return ((x << np.uint32(r)) | (x >> np.uint32(32 - r))).astype(np.uint32)

    k1 = np.uint32(np.uint64(seed) >> np.uint64(32))
    k2 = np.uint32(seed & 0xFFFFFFFF)
    rots = [(13, 15, 26, 6), (17, 29, 16, 24)]
    ks = [k1, k2, np.uint32(k1 ^ k2 ^ np.uint32(0x1BD11BDA))]
    with np.errstate(over="ignore"):
        x0 = np.zeros(size, np.uint32) + ks[0]
        x1 = (np.arange(size, dtype=np.uint32) + ks[1]).astype(np.uint32)
        for i in range(5):
            for r in rots[i % 2]:
                x0 = (x0 + x1).astype(np.uint32)
                x1 = (x0 ^ rotl(x1, r)).astype(np.uint32)
            x0 = (x0 + ks[(i + 1) % 3]).astype(np.uint32)
            x1 = (x1 + ks[(i + 2) % 3] + np.uint32(i + 1)).astype(np.uint32)
    bits = x0 ^ x1
    fb = (bits >> np.uint32(9)) | np.uint32(0x3F800000)
    return np.maximum(np.float32(0.0), fb.view(np.float32) - np.float32(1.0))


_U = _np_uniform_f32(42, B * N)
with np.errstate(divide="ignore"):
    _LU = (np.log(_U) - np.log1p(-_U)).astype(np.float32).reshape(G, BB, N)


def _body(x_ref, lu_ref, pk_ref, w1_ref, w2_ref, out_ref, m_ref):
    g = pl.program_id(0)
    xb = x_ref[...]
    w1 = w1_ref[...]
    w2 = w2_ref[...]
    mt = pk_ref[0:D]
    b1 = pk_ref[D:D + H]
    b2 = pk_ref[D + H:D + H + 1]
    rows = []
    for b in range(BB):
        ht = jnp.dot(w1, xb[b], preferred_element_type=jnp.float32)
        gl = jax.nn.gelu(ht + b1)
        rows.append(jnp.dot(w2, gl, preferred_element_type=jnp.float32))
    s = jnp.concatenate(rows, axis=0) + b2 - lu_ref[g]
    cond = s > 0
    m_ref[g] = cond
    out_ref[...] = jnp.where(cond[:, None, :], mt[None], xb)


def kernel(x, mask_token, W1, b1, W2, b2):
    xt = jnp.transpose(x, (0, 2, 1))
    pk = jnp.concatenate([mask_token.reshape(D), b1, b2]).reshape(D + H + 1, 1)
    outt, m = pl.pallas_call(
        _body,
        grid=(G,),
        in_specs=[
            pl.BlockSpec((BB, D, N), lambda g: (g, 0, 0)),
            pl.BlockSpec((G, BB, N), lambda g: (0, 0, 0)),
            pl.BlockSpec((D + H + 1, 1), lambda g: (0, 0)),
            pl.BlockSpec((H, D), lambda g: (0, 0)),
            pl.BlockSpec((1, H), lambda g: (0, 0)),
        ],
        out_specs=[
            pl.BlockSpec((BB, D, N), lambda g: (g, 0, 0)),
            pl.BlockSpec((G, BB, N), lambda g: (0, 0, 0)),
        ],
        out_shape=[
            jax.ShapeDtypeStruct((B, D, N), jnp.float32),
            jax.ShapeDtypeStruct((G, BB, N), jnp.bool_),
        ],
    )(xt, jnp.asarray(_LU), pk, W1, W2)
    masked = jnp.transpose(outt, (0, 2, 1))
    return masked, m.reshape(B, N)
